# baseline (speedup 1.0000x reference)
import functools

import numpy as np
import jax
import jax.numpy as jnp
from jax import lax
from jax.experimental import pallas as pl
from jax.experimental.pallas import tpu as pltpu
from jax.experimental.pallas import tpu_sc as plsc

_PAIRS = [(2, 7), (1, 13), (2, 68), (1, 110), (2, 129), (1, 207), (2, 190), (1, 304), (2, 251), (1, 401), (2, 312), (1, 498), (2, 373), (1, 595), (2, 434), (1, 692), (2, 495), (1, 789), (2, 556), (1, 886), (2, 617), (1, 983), (2, 678), (1, 1080), (2, 739), (1, 1177), (2, 800), (1, 1274), (2, 861), (1, 1371), (2, 922), (1, 1468), (2, 983), (1, 1565), (2, 1044), (1, 1662), (2, 1105), (1, 1759), (2, 1166), (1, 1856), (2, 1227), (1, 1953), (2, 1288), (1, 2), (2, 1349), (1, 99), (2, 1410), (1, 196), (2, 1471), (1, 293), (2, 1532), (1, 390), (2, 1593), (1, 487), (2, 1654), (1, 584), (2, 1715), (1, 681), (2, 1776), (1, 778), (2, 1837), (1, 875), (2, 1898), (1, 972)]

assert [l for l, _ in _PAIRS] == [2, 1] * 32, "pair list must alternate layer 2/1"

_ROWS = 32
_NW = 32

_IDX2 = np.concatenate(
    [o * _ROWS + np.arange(_ROWS, dtype=np.int32) for l, o in _PAIRS if l == 2]
).astype(np.int32)
_IDX1 = np.concatenate(
    [np.full(_ROWS, o, dtype=np.int32) for l, o in _PAIRS if l == 1]
).astype(np.int32)

_mesh = plsc.VectorSubcoreMesh(core_axis_name="c", subcore_axis_name="s")


@functools.partial(
    pl.kernel,
    out_type=jax.ShapeDtypeStruct((64 * _ROWS, 256), jnp.float32),
    mesh=_mesh,
    scratch_types=[
        pltpu.VMEM((_ROWS,), jnp.int32),
        pltpu.VMEM((_ROWS,), jnp.int32),
        pltpu.VMEM((_ROWS, 256), jnp.float32),
        pltpu.VMEM((_ROWS, 256), jnp.float32),
        pltpu.SemaphoreType.DMA,
        pltpu.SemaphoreType.DMA,
    ],
)
def _gather(l2_hbm, l1_hbm, idx2_hbm, idx1_hbm, out_hbm,
            idx2_v, idx1_v, rows2_v, rows1_v, sem2, sem1):
    w = lax.axis_index("s") * 2 + lax.axis_index("c")
    pltpu.sync_copy(idx2_hbm.at[pl.ds(w * _ROWS, _ROWS)], idx2_v)
    pltpu.sync_copy(idx1_hbm.at[pl.ds(w * _ROWS, _ROWS)], idx1_v)
    c2 = pltpu.async_copy(l2_hbm.at[idx2_v], rows2_v, sem2)
    c1 = pltpu.async_copy(l1_hbm.at[idx1_v], rows1_v, sem1)
    c2.wait()
    c1.wait()
    pltpu.sync_copy(rows2_v, out_hbm.at[pl.ds(w * 2 * _ROWS, _ROWS)])
    pltpu.sync_copy(rows1_v, out_hbm.at[pl.ds(w * 2 * _ROWS + _ROWS, _ROWS)])


@jax.jit
def kernel(layer2, layer1):
    l2 = layer2.reshape(2048 * _ROWS, 256)
    l1 = layer1.reshape(2048, 256)
    out = _gather(l2, l1, jnp.asarray(_IDX2), jnp.asarray(_IDX1))
    return out.reshape(64, _ROWS, 256)
